# baseline (speedup 1.0000x reference)
import jax
import jax.numpy as jnp
import numpy as np
from jax.experimental import pallas as pl
from jax.experimental.pallas import tpu as pltpu

_NCHUNK = 8


def _body(maskr, feat_h, pack_h, flat_h, epsT_h, adj_hbm,
          act_o, val_o,
          feat, pack, flat, epsT, a_vmem, sem_in, sem_adj):
    f32 = jnp.float32
    n = a_vmem.shape[0]
    rows = n // _NCHUNK

    ins = [(feat_h, feat), (pack_h, pack), (flat_h, flat), (epsT_h, epsT)]

    def in_cp(i):
        return pltpu.make_async_copy(ins[i][0], ins[i][1], sem_in.at[i])

    def adj_cp(k):
        return pltpu.make_async_copy(
            adj_hbm.at[pl.ds(k * rows, rows), :],
            a_vmem.at[pl.ds(k * rows, rows), :],
            sem_adj.at[k])

    for i in range(len(ins)):
        in_cp(i).start()
    for k in range(_NCHUNK):
        adj_cp(k).start()
    in_cp(0).wait()
    in_cp(1).wait()

    We1 = pack[0:256, :]
    We2 = pack[256:288, :]
    Wg = pack[288:320, :]
    Wgd = pack[320:352, :]
    Wp1a = pack[352:384, :]
    Wp1b = pack[384:416, :]
    Wp2 = pack[416:448, :]

    in_cp(2).wait()
    be1 = flat[:, 0:32]
    be2 = flat[:, 32:64]
    X1 = jax.nn.relu(jnp.dot(feat[:], We1, preferred_element_type=f32) + be1)
    Xn = jax.nn.relu(jnp.dot(X1, We2, preferred_element_type=f32) + be2)
    Xln = jnp.dot(Xn, Wg, preferred_element_type=f32)
    XT = Xn.T
    XlT = Xln.T
    deg = jnp.full((1, n), 1.0, f32)
    for k in range(_NCHUNK):
        adj_cp(k).wait()
        deg = deg + jnp.sum(a_vmem[pl.ds(k * rows, rows), :], axis=0, keepdims=True)
    dinv = jnp.where(deg > 0, 1.0 / jnp.sqrt(deg), 0.0)
    ST = XlT * dinv
    Y0T = jax.lax.dot_general(ST, a_vmem[:], (((1,), (0,)), ((), ())),
                              precision=jax.lax.Precision.DEFAULT,
                              preferred_element_type=f32)
    YT = Y0T * dinv + XlT * (dinv * dinv)
    bg = flat[:, 64:96]
    bgd = flat[:, 96:128]
    bp1 = flat[:, 128:160]
    bp2 = flat[:, 160:192]
    XgT = jax.nn.relu(YT + bg.T)
    Xg2T = jax.nn.relu(jnp.dot(Wgd.T, XgT, preferred_element_type=f32) + bgd.T)
    XpT = jax.nn.relu(jnp.dot(Wp1a.T, Xg2T, preferred_element_type=f32)
                      + jnp.dot(Wp1b.T, XT, preferred_element_type=f32)
                      + bp1.T)
    XpT = jax.nn.relu(jnp.dot(Wp2.T, XpT, preferred_element_type=f32) + bp2.T)
    WhT = jnp.concatenate([
        flat[:, 198:230],
        flat[:, 230:262], flat[:, 262:294],
        flat[:, 294:326], flat[:, 326:358], flat[:, 358:390],
    ], axis=0)
    bh = flat[:, 192:198]
    HT = jnp.dot(WhT, XpT, preferred_element_type=f32) + bh.T
    in_cp(3).wait()
    val_o[:] = HT[0:1, :].T
    muT = jnp.tanh(HT[1:3, :])
    zT = jnp.tanh(HT[3:6, :])
    sigT = jnp.concatenate([jnp.exp(-zT[0:1, :]), jnp.exp(-zT[2:3, :])], axis=0)
    act = jnp.clip(muT + epsT[:] * sigT, -1.0, 1.0) * maskr[:]
    act_o[:] = act.T


def kernel(features, adjacency, mask, We1, be1, We2, be2, Wg, bg, Wgd, bgd,
           Wp1, bp1, Wp2, bp2, Wv, bv, Wmu, bmu, WL, bL):
    n, fdim = features.shape
    A = Wmu.shape[1]
    epsT = jax.random.normal(jax.random.key(42), (n, A), jnp.float32).T
    pack = jnp.concatenate(
        [We1, We2, Wg, Wgd, Wp1[0:32], Wp1[32:64], Wp2], axis=0)
    flat = jnp.concatenate(
        [be1, be2, bg, bgd, bp1, bp2,
         bv, bmu, bL,
         Wv.T.reshape(-1), Wmu.T.reshape(-1), WL.T.reshape(-1)],
        axis=0).reshape(1, -1)
    hbm = pl.BlockSpec(memory_space=pltpu.MemorySpace.HBM)
    vmem = pl.BlockSpec(memory_space=pltpu.MemorySpace.VMEM)
    act, val = pl.pallas_call(
        _body,
        in_specs=[vmem, hbm, hbm, hbm, hbm, hbm],
        out_shape=(
            jax.ShapeDtypeStruct((n, A), jnp.float32),
            jax.ShapeDtypeStruct((n, 1), jnp.float32),
        ),
        scratch_shapes=[
            pltpu.MemorySpace.VMEM(features.shape, jnp.float32),
            pltpu.MemorySpace.VMEM(pack.shape, jnp.float32),
            pltpu.MemorySpace.VMEM(flat.shape, jnp.float32),
            pltpu.MemorySpace.VMEM(epsT.shape, jnp.float32),
            pltpu.MemorySpace.VMEM((n, n), jnp.float32),
            pltpu.SemaphoreType.DMA((4,)),
            pltpu.SemaphoreType.DMA((_NCHUNK,)),
        ],
    )(mask.reshape(1, n), features, pack, flat, epsT, adjacency)
    return (act, val)
